# baseline (speedup 1.0000x reference)
import functools

import jax
import jax.numpy as jnp
from jax import lax
from jax.experimental import pallas as pl
from jax.experimental.pallas import tpu as pltpu
from jax.experimental.pallas import tpu_sc as plsc

def _tc_body(x1_ref, x2_ref, w_ref, b_ref, o_ref, *, dh):
    acc = jnp.dot(x1_ref[...], w_ref[pl.ds(0, dh), :],
                  preferred_element_type=jnp.float32)
    acc += jnp.dot(x2_ref[...], w_ref[pl.ds(dh, dh), :],
                   preferred_element_type=jnp.float32)
    o_ref[...] = acc + b_ref[...]


def _heads_matmul(x, w_all, b_all, bn, row0, nrows, ec):
    n, d = x.shape
    blk0 = row0 // bn
    dh = d // 2
    body = functools.partial(_tc_body, dh=dh)
    return pl.pallas_call(
        body,
        grid=(nrows // bn,),
        in_specs=[
            pl.BlockSpec((bn, dh), lambda i: (i + blk0, 0)),
            pl.BlockSpec((bn, dh), lambda i: (i + blk0, 1)),
            pl.BlockSpec((d, ec), lambda i: (0, 0)),
            pl.BlockSpec((1, ec), lambda i: (0, 0)),
        ],
        out_specs=pl.BlockSpec((bn, ec), lambda i: (i, 0)),
        out_shape=jax.ShapeDtypeStruct((nrows, ec), jnp.float32),
    )(x, x, w_all, b_all)


def _sc_route(full, t1d, n, c, ec):
    info = plsc.get_sparse_core_info()
    nc, ns = info.num_cores, info.num_subcores
    nw = nc * ns
    per_w = n // nw
    nelem = per_w * c
    ngroup = nelem // 16

    mesh = plsc.VectorSubcoreMesh(core_axis_name="c", subcore_axis_name="s")

    @functools.partial(
        pl.kernel,
        out_type=jax.ShapeDtypeStruct((n, c), jnp.float32),
        mesh=mesh,
        compiler_params=pltpu.CompilerParams(needs_layout_passes=False),
        scratch_types=[
            pltpu.VMEM((per_w,), jnp.int32),
            pltpu.VMEM((per_w, ec), jnp.float32),
            pltpu.VMEM((per_w, c), jnp.float32),
        ],
    )
    def k(full_ref, t_ref, out_ref, tv, chunk, outv):
        wid = lax.axis_index("s") * nc + lax.axis_index("c")
        base = wid * per_w
        pltpu.sync_copy(t_ref.at[pl.ds(base, per_w)], tv)
        pltpu.sync_copy(full_ref.at[pl.ds(base, per_w)], chunk)
        lane = lax.broadcasted_iota(jnp.int32, (16,), 0)

        def body(tg, _):
            tok = tg * 16 + lane
            colbase = tv[pl.ds(tg * 16, 16)] * c
            for cc in range(c):
                vals = plsc.load_gather(chunk, [tok, colbase + cc])
                plsc.store_scatter(outv, [tok, jnp.full((16,), cc, jnp.int32)], vals)
            return 0

        lax.fori_loop(0, per_w // 16, body, 0, unroll=1)
        pltpu.sync_copy(outv, out_ref.at[pl.ds(base, per_w)])

    return k(full, t1d)


def kernel(x, t, W, b):
    n, d = x.shape
    e, _, c = W.shape
    ec = e * c
    w_all = W.transpose(1, 0, 2).reshape(d, ec)
    b_all = b.reshape(1, ec)
    t1d = t.astype(jnp.int32)
    full = _heads_matmul(x, w_all, b_all, 2048, 0, n, ec)
    return _sc_route(full, t1d, n, c, ec)


# speedup vs baseline: 1.0124x; 1.0124x over previous
import functools

import jax
import jax.numpy as jnp
from jax import lax
from jax.experimental import pallas as pl
from jax.experimental.pallas import tpu as pltpu
from jax.experimental.pallas import tpu_sc as plsc

def _tc_body(x_ref, w_ref, b_ref, o_ref):
    o_ref[...] = (
        jnp.dot(x_ref[...], w_ref[...], preferred_element_type=jnp.float32)
        + b_ref[...]
    )


def _heads_matmul(x, w_all, b_all, bn, row0, nrows, ec):
    n, d = x.shape
    blk0 = row0 // bn
    return pl.pallas_call(
        _tc_body,
        grid=(nrows // bn,),
        in_specs=[
            pl.BlockSpec((bn, d), lambda i: (i + blk0, 0)),
            pl.BlockSpec((d, ec), lambda i: (0, 0)),
            pl.BlockSpec((1, ec), lambda i: (0, 0)),
        ],
        out_specs=pl.BlockSpec((bn, ec), lambda i: (i, 0)),
        out_shape=jax.ShapeDtypeStruct((nrows, ec), jnp.float32),
    )(x, w_all, b_all)


def _sc_route(full, t1d, n, c, ec):
    info = plsc.get_sparse_core_info()
    nc, ns = info.num_cores, info.num_subcores
    nw = nc * ns
    per_w = n // nw
    nelem = per_w * c
    ngroup = nelem // 16

    mesh = plsc.VectorSubcoreMesh(core_axis_name="c", subcore_axis_name="s")

    @functools.partial(
        pl.kernel,
        out_type=jax.ShapeDtypeStruct((n, c), jnp.float32),
        mesh=mesh,
        compiler_params=pltpu.CompilerParams(needs_layout_passes=False),
        scratch_types=[
            pltpu.VMEM((per_w,), jnp.int32),
            pltpu.VMEM((per_w, ec), jnp.float32),
            pltpu.VMEM((per_w, c), jnp.float32),
            pltpu.SemaphoreType.DMA,
            pltpu.SemaphoreType.DMA,
        ],
    )
    def k(full_ref, t_ref, out_ref, tv, chunk, outv, sem0, sem1):
        wid = lax.axis_index("s") * nc + lax.axis_index("c")
        base = wid * per_w
        half = per_w // 2
        cp_t = pltpu.async_copy(t_ref.at[pl.ds(base, per_w)], tv, sem0)
        cp0 = pltpu.async_copy(
            full_ref.at[pl.ds(base, half)], chunk.at[pl.ds(0, half)], sem0
        )
        cp1 = pltpu.async_copy(
            full_ref.at[pl.ds(base + half, half)], chunk.at[pl.ds(half, half)], sem1
        )
        lane = lax.broadcasted_iota(jnp.int32, (16,), 0)

        def body(tg, _):
            tok = tg * 16 + lane
            colbase = tv[pl.ds(tg * 16, 16)] * c
            for cc in range(c):
                vals = plsc.load_gather(chunk, [tok, colbase + cc])
                plsc.store_scatter(outv, [tok, jnp.full((16,), cc, jnp.int32)], vals)
            return 0

        cp_t.wait()
        cp0.wait()
        lax.fori_loop(0, half // 16, body, 0, unroll=1)
        cp1.wait()
        lax.fori_loop(half // 16, per_w // 16, body, 0, unroll=1)
        pltpu.sync_copy(outv, out_ref.at[pl.ds(base, per_w)])

    return k(full, t1d)


def kernel(x, t, W, b):
    n, d = x.shape
    e, _, c = W.shape
    ec = e * c
    w_all = W.transpose(1, 0, 2).reshape(d, ec)
    b_all = b.reshape(1, ec)
    t1d = t.astype(jnp.int32)
    full = _heads_matmul(x, w_all, b_all, 2048, 0, n, ec)
    return _sc_route(full, t1d, n, c, ec)


# speedup vs baseline: 1.0171x; 1.0047x over previous
import functools

import jax
import jax.numpy as jnp
from jax import lax
from jax.experimental import pallas as pl
from jax.experimental.pallas import tpu as pltpu
from jax.experimental.pallas import tpu_sc as plsc

def _tc_body(x_ref, w_ref, b_ref, o_ref):
    o_ref[...] = (
        jnp.dot(x_ref[...], w_ref[...], preferred_element_type=jnp.float32)
        + b_ref[...]
    )


def _heads_matmul(x, w_all, b_all, bn, row0, nrows, ec):
    n, d = x.shape
    blk0 = row0 // bn
    return pl.pallas_call(
        _tc_body,
        grid=(nrows // bn,),
        in_specs=[
            pl.BlockSpec((bn, d), lambda i: (i + blk0, 0)),
            pl.BlockSpec((d, ec), lambda i: (0, 0)),
            pl.BlockSpec((1, ec), lambda i: (0, 0)),
        ],
        out_specs=pl.BlockSpec((bn, ec), lambda i: (i, 0)),
        out_shape=jax.ShapeDtypeStruct((nrows, ec), jnp.float32),
    )(x, w_all, b_all)


def _sc_route(full, t1d, n, c, ec):
    info = plsc.get_sparse_core_info()
    nc, ns = info.num_cores, info.num_subcores
    nw = nc * ns
    per_w = n // nw

    mesh = plsc.VectorSubcoreMesh(core_axis_name="c", subcore_axis_name="s")

    @functools.partial(
        pl.kernel,
        out_type=jax.ShapeDtypeStruct((n, c), jnp.float32),
        mesh=mesh,
        compiler_params=pltpu.CompilerParams(needs_layout_passes=False),
        scratch_types=[
            pltpu.VMEM((per_w,), jnp.int32),
            pltpu.VMEM((per_w, ec), jnp.float32),
            pltpu.VMEM((per_w, c), jnp.float32),
            pltpu.SemaphoreType.DMA,
            pltpu.SemaphoreType.DMA,
        ],
    )
    def k(full_ref, t_ref, out_ref, tv, chunk, outv, sem0, sem1):
        wid = lax.axis_index("s") * nc + lax.axis_index("c")
        base = wid * per_w
        half = per_w // 2
        cp_t = pltpu.async_copy(t_ref.at[pl.ds(base, per_w)], tv, sem0)
        cp0 = pltpu.async_copy(
            full_ref.at[pl.ds(base, half)], chunk.at[pl.ds(0, half)], sem0
        )
        cp1 = pltpu.async_copy(
            full_ref.at[pl.ds(base + half, half)], chunk.at[pl.ds(half, half)], sem1
        )
        lane = lax.broadcasted_iota(jnp.int32, (16,), 0)

        def body(tg, _):
            tok = tg * 16 + lane
            colbase = tv[pl.ds(tg * 16, 16)] * c
            for cc in range(c):
                vals = plsc.load_gather(chunk, [tok, colbase + cc])
                plsc.store_scatter(outv, [tok, jnp.full((16,), cc, jnp.int32)], vals)
            return 0

        cp_t.wait()
        cp0.wait()
        lax.fori_loop(0, half // 16, body, 0, unroll=8)
        cp_o0 = pltpu.async_copy(
            outv.at[pl.ds(0, half)], out_ref.at[pl.ds(base, half)], sem0
        )
        cp1.wait()
        lax.fori_loop(half // 16, per_w // 16, body, 0, unroll=8)
        cp_o0.wait()
        pltpu.sync_copy(
            outv.at[pl.ds(half, half)], out_ref.at[pl.ds(base + half, half)]
        )

    return k(full, t1d)


def kernel(x, t, W, b):
    n, d = x.shape
    e, _, c = W.shape
    ec = e * c
    w_all = W.transpose(1, 0, 2).reshape(d, ec)
    b_all = b.reshape(1, ec)
    t1d = t.astype(jnp.int32)
    full = _heads_matmul(x, w_all, b_all, 2048, 0, n, ec)
    return _sc_route(full, t1d, n, c, ec)
